# Initial kernel scaffold; baseline (speedup 1.0000x reference)
import jax
import jax.numpy as jnp
from jax.experimental import pallas as pl


def kernel(features, edge_index, edgesAttr, adjacency, node2node_features, params):
    raise NotImplementedError("write your pallas kernel here")


# speedup vs baseline: 12.4510x; 12.4510x over previous
import jax
import jax.numpy as jnp
from jax import lax
from jax.experimental import pallas as pl
from jax.experimental.pallas import tpu as pltpu

_N = 1024
_HID = 128
_NH = 16
_HH = 256
_D1 = _NH * _HH
_ALPHA = 0.2
_BLK = 256
_NB = _N // _BLK
_NEG = -1e9


def _sig(x):
  return 1.0 / (1.0 + jnp.exp(-x))


def _leaky(x):
  return jnp.where(x > 0, x, _ALPHA * x)


def _elu(x):
  return jnp.where(x > 0, x, jnp.exp(x) - 1.0)


def _k1_body(feat_ref, snw_ref, sna_ref, g1w_ref, g1b_ref, w1c_ref,
             wh_ref, num_ref, den_ref):
  i = pl.program_id(0)
  f = feat_ref[...]
  h0 = jnp.dot(f, snw_ref[...], preferred_element_type=jnp.float32)
  t = jnp.dot(h0, sna_ref[...], preferred_element_type=jnp.float32)
  h = _elu(_sig(_leaky(t)) * h0)
  sg = _sig(jnp.dot(h, g1w_ref[...], preferred_element_type=jnp.float32)
            + g1b_ref[0, 0])
  e = jnp.exp(sg)
  wh_ref[...] = jnp.dot(h, w1c_ref[...], preferred_element_type=jnp.float32)

  @pl.when(i == 0)
  def _():
    num_ref[...] = jnp.zeros_like(num_ref)
    den_ref[...] = jnp.zeros_like(den_ref)

  num_ref[...] += jnp.sum(e * h, axis=0, keepdims=True)
  den_ref[...] += jnp.sum(e, axis=0, keepdims=True)


def _k2_body(whh_ref, whb_ref, adj_ref, n2n_ref, asrc_ref, adst_ref, epg_ref,
             ae_ref, x_ref, uu3_ref):
  h = pl.program_id(1)
  whh = whh_ref[...]
  st = jnp.dot(whb_ref[...], asrc_ref[...],
               preferred_element_type=jnp.float32)
  dt = jnp.reshape(jnp.dot(whh, adst_ref[...],
                           preferred_element_type=jnp.float32), (1, _N))
  v = n2n_ref[...]
  esc = (ae_ref[0, h] * v[0] + ae_ref[1, h] * v[1]
         + ae_ref[2, h] * v[2] + ae_ref[3, h] * v[3])
  sc = _leaky(st + dt + esc)
  sc = jnp.where(adj_ref[...] > 0, sc, _NEG)
  m = jnp.max(sc, axis=1, keepdims=True)
  p = jnp.exp(sc - m)
  att = p / jnp.sum(p, axis=1, keepdims=True)
  o = _elu(jnp.dot(att, whh, preferred_element_type=jnp.float32))
  x_ref[...] = o
  c = jnp.dot(o, epg_ref[...], preferred_element_type=jnp.float32)

  @pl.when(h == 0)
  def _():
    uu3_ref[...] = jnp.zeros_like(uu3_ref)

  uu3_ref[...] += c


def _k4_body(uu3_ref, adj_ref, ep1b_ref, alpha_ref):
  u1 = uu3_ref[:, 0:1]
  u2r = jnp.reshape(uu3_ref[:, 1:2], (1, _N))
  s = _sig(u1 + u2r + ep1b_ref[0, 0])
  ns = jnp.sum(adj_ref[...] * s, axis=0, keepdims=True)
  alpha_ref[...] = jnp.reshape(1.0 + jnp.tanh(ns), (_N, 1))


def _k4c_body(x_ref, al_ref, uu3_ref, w2_ref, g2b_ref,
              wh2_ref, num_ref, den_ref):
  i = pl.program_id(0)
  al = al_ref[...]
  xn = al * x_ref[...]
  sg = _sig(al * uu3_ref[:, 2:3] + g2b_ref[0, 0])
  e = jnp.exp(sg)
  wh2_ref[...] = jnp.dot(xn, w2_ref[...], preferred_element_type=jnp.float32)

  @pl.when(i == 0)
  def _():
    num_ref[...] = jnp.zeros_like(num_ref)
    den_ref[...] = jnp.zeros_like(den_ref)

  num_ref[...] += jnp.sum(e * xn, axis=0, keepdims=True)
  den_ref[...] += jnp.sum(e, axis=0, keepdims=True)


def _k5_body(wh2f_ref, wh2b_ref, adj_ref, n2n_ref, uu3f_ref, uu3b_ref,
             asrc2_ref, adst2_ref, we_ref, ae2_ref, ep1b_ref, out_ref):
  wh2 = wh2f_ref[...]
  st2 = jnp.dot(wh2b_ref[...], asrc2_ref[...],
                preferred_element_type=jnp.float32)
  dt2 = jnp.reshape(jnp.dot(wh2, adst2_ref[...],
                            preferred_element_type=jnp.float32), (1, _N))
  u1 = uu3b_ref[:, 0:1]
  u2r = jnp.reshape(uu3f_ref[:, 1:2], (1, _N))
  s = _sig(u1 + u2r + ep1b_ref[0, 0])
  v = n2n_ref[...]
  acc = jnp.zeros((_BLK, _N), jnp.float32)
  for k in range(_NH):
    t = (we_ref[0, k] * v[0] + we_ref[1, k] * v[1]
         + we_ref[2, k] * v[2] + we_ref[3, k] * v[3])
    acc += ae2_ref[0, k] * _elu(t)
  sc = _leaky(st2 + dt2 + s * acc)
  sc = jnp.where(adj_ref[...] > 0, sc, _NEG)
  m = jnp.max(sc, axis=1, keepdims=True)
  p = jnp.exp(sc - m)
  att = p / jnp.sum(p, axis=1, keepdims=True)
  out_ref[...] = jnp.dot(att, wh2, preferred_element_type=jnp.float32)


def _k6_body(x2_ref, adj_ref, ep2w_ref, g3w_ref, n0_ref, d1_ref, n1_ref,
             d2_ref, wfa_ref, wfb_ref, wfc_ref, wba_ref, wbb_ref, wbc_ref,
             bf_ref, bb_ref, w2f_ref, w2b_ref, b2f_ref, b2b_ref,
             fcw_ref, fcb_ref, ep2b_ref, g3b_ref, out_ref):
  x2 = x2_ref[...]
  uu = jnp.dot(x2, ep2w_ref[...], preferred_element_type=jnp.float32)
  u1 = uu[:, 0:1]
  u2r = jnp.reshape(uu[:, 1:2], (1, _N))
  s2 = _sig(u1 + u2r + ep2b_ref[0, 0])
  ns2 = jnp.sum(adj_ref[...] * s2, axis=0, keepdims=True)
  al2 = jnp.reshape(1.0 + jnp.tanh(ns2), (_N, 1))
  xn2 = al2 * x2
  sg = _sig(jnp.dot(xn2, g3w_ref[...], preferred_element_type=jnp.float32)
            + g3b_ref[0, 0])
  m = jnp.max(sg)
  p = jnp.exp(sg - m)
  g = p / jnp.sum(p)
  hs2 = jnp.sum(g * xn2, axis=0, keepdims=True)
  hs0 = n0_ref[...] / d1_ref[...]
  hs1 = n1_ref[...] / d2_ref[...]

  def gate(z):
    i_ = z[:, 0:128]
    f_ = z[:, 128:256]
    g_ = z[:, 256:384]
    o_ = z[:, 384:512]
    del f_
    c = _sig(i_) * jnp.tanh(g_)
    return _sig(o_) * jnp.tanh(c)

  def lin3(wa, wb, wc, b):
    return (jnp.dot(hs0, wa, preferred_element_type=jnp.float32)
            + jnp.dot(hs1, wb, preferred_element_type=jnp.float32)
            + jnp.dot(hs2, wc, preferred_element_type=jnp.float32)
            + b)

  hf = gate(lin3(wfa_ref[...], wfb_ref[...], wfc_ref[...], bf_ref[...]))
  hb = gate(lin3(wba_ref[...], wbb_ref[...], wbc_ref[...], bb_ref[...]))

  def lin2(w, b):
    return (jnp.dot(hf, w[0:128, :], preferred_element_type=jnp.float32)
            + jnp.dot(hb, w[128:256, :], preferred_element_type=jnp.float32)
            + b)

  hf2 = gate(lin2(w2f_ref[...], b2f_ref[...]))
  hb2 = gate(lin2(w2b_ref[...], b2b_ref[...]))
  fw = fcw_ref[...]
  logits = (jnp.dot(hf2, fw[0:128, :], preferred_element_type=jnp.float32)
            + jnp.dot(hb2, fw[128:256, :], preferred_element_type=jnp.float32)
            + fcb_ref[...])
  m2 = jnp.max(logits, axis=1, keepdims=True)
  e2 = jnp.exp(logits - m2)
  out_ref[...] = e2 / jnp.sum(e2, axis=1, keepdims=True)


def _vspec(block=None, index_map=None):
  if block is None:
    return pl.BlockSpec(memory_space=pltpu.ANY)
  return pl.BlockSpec(block, index_map)


_SMEM = pl.BlockSpec(memory_space=pltpu.SMEM)
_F32 = jnp.float32


def kernel(features, edge_index, edgesAttr, adjacency, node2node_features,
           params):
  del edge_index, edgesAttr
  P = params
  zero = lambda *_: (0, 0)

  w1cat = jnp.transpose(P['W1'], (1, 0, 2)).reshape(_HID, _D1)
  asrc1 = P['a_src1'][:, :, 0].reshape(_D1, 1)
  adst1 = P['a_dst1'][:, :, 0].reshape(_D1, 1)
  ae1 = jnp.transpose(P['a_e1'][:, :, 0])
  we1c = jnp.transpose(P['We1'][:, :, 0])
  ae2 = jnp.transpose(P['a_e2'])
  epg = jnp.concatenate(
      [P['ep1_w'][:_D1], P['ep1_w'][_D1:], P['g2_w'],
       jnp.zeros((_D1, 5), _F32)], axis=1)
  ep2w = jnp.concatenate([P['ep2_w'][:_HH], P['ep2_w'][_HH:]], axis=1)
  n2nT = jnp.transpose(node2node_features.reshape(_N, _N, 4), (2, 0, 1))
  g1b = P['g1_b'].reshape(1, 1)
  g2b = P['g2_b'].reshape(1, 1)
  g3b = P['g3_b'].reshape(1, 1)
  ep1b = P['ep1_b'].reshape(1, 1)
  ep2b = P['ep2_b'].reshape(1, 1)
  bf = P['l1f_b'].reshape(1, 512)
  bb = P['l1b_b'].reshape(1, 512)
  b2f = P['l2f_b'].reshape(1, 512)
  b2b = P['l2b_b'].reshape(1, 512)
  fcb = P['fc_b'].reshape(1, 2)
  wfa, wfb, wfc = (P['l1f_Wi'][:128], P['l1f_Wi'][128:128 + _D1],
                   P['l1f_Wi'][128 + _D1:])
  wba, wbb, wbc = (P['l1b_Wi'][:128], P['l1b_Wi'][128:128 + _D1],
                   P['l1b_Wi'][128 + _D1:])

  wh, hs0num, den1 = pl.pallas_call(
      _k1_body,
      grid=(_NB,),
      in_specs=[
          pl.BlockSpec((_BLK, _HID), lambda i: (i, 0)),
          pl.BlockSpec((_HID, _HID), zero),
          pl.BlockSpec((_HID, 1), zero),
          pl.BlockSpec((_HID, 1), zero),
          _SMEM,
          pl.BlockSpec((_HID, _D1), zero),
      ],
      out_specs=[
          pl.BlockSpec((_BLK, _D1), lambda i: (i, 0)),
          pl.BlockSpec((1, _HID), zero),
          pl.BlockSpec((1, 1), zero),
      ],
      out_shape=[
          jax.ShapeDtypeStruct((_N, _D1), _F32),
          jax.ShapeDtypeStruct((1, _HID), _F32),
          jax.ShapeDtypeStruct((1, 1), _F32),
      ],
  )(features, P['snal_W'], P['snal_a'], P['g1_w'], g1b, w1cat)

  x, uu3 = pl.pallas_call(
      _k2_body,
      grid=(_NB, _NH),
      in_specs=[
          pl.BlockSpec((_N, _HH), lambda i, h: (0, h)),
          pl.BlockSpec((_BLK, _HH), lambda i, h: (i, h)),
          pl.BlockSpec((_BLK, _N), lambda i, h: (i, 0)),
          pl.BlockSpec((4, _BLK, _N), lambda i, h: (0, i, 0)),
          pl.BlockSpec((_HH, 1), lambda i, h: (h, 0)),
          pl.BlockSpec((_HH, 1), lambda i, h: (h, 0)),
          pl.BlockSpec((_HH, 8), lambda i, h: (h, 0)),
          _SMEM,
      ],
      out_specs=[
          pl.BlockSpec((_BLK, _HH), lambda i, h: (i, h)),
          pl.BlockSpec((_BLK, 8), lambda i, h: (i, 0)),
      ],
      out_shape=[
          jax.ShapeDtypeStruct((_N, _D1), _F32),
          jax.ShapeDtypeStruct((_N, 8), _F32),
      ],
  )(wh, wh, adjacency, n2nT, asrc1, adst1, epg, ae1)

  alpha = pl.pallas_call(
      _k4_body,
      in_specs=[
          pl.BlockSpec((_N, 8), zero),
          pl.BlockSpec((_N, _N), zero),
          _SMEM,
      ],
      out_specs=pl.BlockSpec((_N, 1), zero),
      out_shape=jax.ShapeDtypeStruct((_N, 1), _F32),
  )(uu3, adjacency, ep1b)

  wh2, hs1num, den2 = pl.pallas_call(
      _k4c_body,
      grid=(_NB,),
      in_specs=[
          pl.BlockSpec((_BLK, _D1), lambda i: (i, 0)),
          pl.BlockSpec((_BLK, 1), lambda i: (i, 0)),
          pl.BlockSpec((_BLK, 8), lambda i: (i, 0)),
          pl.BlockSpec((_D1, _HH), lambda i: (0, 0)),
          _SMEM,
      ],
      out_specs=[
          pl.BlockSpec((_BLK, _HH), lambda i: (i, 0)),
          pl.BlockSpec((1, _D1), lambda i: (0, 0)),
          pl.BlockSpec((1, 1), lambda i: (0, 0)),
      ],
      out_shape=[
          jax.ShapeDtypeStruct((_N, _HH), _F32),
          jax.ShapeDtypeStruct((1, _D1), _F32),
          jax.ShapeDtypeStruct((1, 1), _F32),
      ],
  )(x, alpha, uu3, P['W2'], g2b)

  hout2 = pl.pallas_call(
      _k5_body,
      grid=(_NB,),
      in_specs=[
          pl.BlockSpec((_N, _HH), lambda i: (0, 0)),
          pl.BlockSpec((_BLK, _HH), lambda i: (i, 0)),
          pl.BlockSpec((_BLK, _N), lambda i: (i, 0)),
          pl.BlockSpec((4, _BLK, _N), lambda i: (0, i, 0)),
          pl.BlockSpec((_N, 8), lambda i: (0, 0)),
          pl.BlockSpec((_BLK, 8), lambda i: (i, 0)),
          pl.BlockSpec((_HH, 1), lambda i: (0, 0)),
          pl.BlockSpec((_HH, 1), lambda i: (0, 0)),
          _SMEM,
          _SMEM,
          _SMEM,
      ],
      out_specs=pl.BlockSpec((_BLK, _HH), lambda i: (i, 0)),
      out_shape=jax.ShapeDtypeStruct((_N, _HH), _F32),
  )(wh2, wh2, adjacency, n2nT, uu3, uu3, P['a_src2'], P['a_dst2'],
    we1c, ae2, ep1b)

  out = pl.pallas_call(
      _k6_body,
      in_specs=[
          pl.BlockSpec((_N, _HH), zero),
          pl.BlockSpec((_N, _N), zero),
          pl.BlockSpec((_HH, 2), zero),
          pl.BlockSpec((_HH, 1), zero),
          pl.BlockSpec((1, _HID), zero),
          pl.BlockSpec((1, 1), zero),
          pl.BlockSpec((1, _D1), zero),
          pl.BlockSpec((1, 1), zero),
          pl.BlockSpec((128, 512), zero),
          pl.BlockSpec((_D1, 512), zero),
          pl.BlockSpec((_HH, 512), zero),
          pl.BlockSpec((128, 512), zero),
          pl.BlockSpec((_D1, 512), zero),
          pl.BlockSpec((_HH, 512), zero),
          pl.BlockSpec((1, 512), zero),
          pl.BlockSpec((1, 512), zero),
          pl.BlockSpec((_HH, 512), zero),
          pl.BlockSpec((_HH, 512), zero),
          pl.BlockSpec((1, 512), zero),
          pl.BlockSpec((1, 512), zero),
          pl.BlockSpec((_HH, 2), zero),
          pl.BlockSpec((1, 2), zero),
          _SMEM,
          _SMEM,
      ],
      out_specs=pl.BlockSpec((1, 2), zero),
      out_shape=jax.ShapeDtypeStruct((1, 2), _F32),
  )(hout2, adjacency, ep2w, P['g3_w'], hs0num, den1, hs1num, den2,
    wfa, wfb, wfc, wba, wbb, wbc, bf, bb, P['l2f_Wi'], P['l2b_Wi'],
    b2f, b2b, P['fc_w'], fcb, ep2b, g3b)

  return out.reshape(2)


# speedup vs baseline: 14.7479x; 1.1845x over previous
import jax
import jax.numpy as jnp
from jax import lax
from jax.experimental import pallas as pl
from jax.experimental.pallas import tpu as pltpu

_N = 1024
_HID = 128
_NH = 16
_HH = 256
_D1 = _NH * _HH
_ALPHA = 0.2
_BLK = 256
_NB = _N // _BLK
_NEG = -1e9


def _sig(x):
  return 1.0 / (1.0 + jnp.exp(-x))


def _leaky(x):
  return jnp.where(x > 0, x, _ALPHA * x)


def _elu(x):
  return jnp.where(x > 0, x, jnp.exp(x) - 1.0)


def _k1_body(feat_ref, snw_ref, sna_ref, g1w_ref, g1b_ref, w1c_ref,
             asrc_ref, adst_ref, whb_ref, stall_ref, dtall_ref,
             num_ref, den_ref):
  i = pl.program_id(0)
  f = feat_ref[...]
  h0 = jnp.dot(f, snw_ref[...], preferred_element_type=jnp.float32)
  t = jnp.dot(h0, sna_ref[...], preferred_element_type=jnp.float32)
  h = _elu(_sig(_leaky(t)) * h0)
  sg = _sig(jnp.dot(h, g1w_ref[...], preferred_element_type=jnp.float32)
            + g1b_ref[0, 0])
  e = jnp.exp(sg)
  wh = jnp.dot(h, w1c_ref[...], preferred_element_type=jnp.float32)
  whb_ref[...] = wh.astype(jnp.bfloat16)
  stall_ref[...] = jnp.dot(wh, asrc_ref[...],
                           preferred_element_type=jnp.float32)
  dtall_ref[...] = jnp.dot(wh, adst_ref[...],
                           preferred_element_type=jnp.float32)

  @pl.when(i == 0)
  def _():
    num_ref[...] = jnp.zeros_like(num_ref)
    den_ref[...] = jnp.zeros_like(den_ref)

  num_ref[...] += jnp.sum(e * h, axis=0, keepdims=True)
  den_ref[...] += jnp.sum(e, axis=0, keepdims=True)


def _k2_body(whbf_ref, stall_ref, dtt_ref, adj_ref, n2n_ref, epg_ref,
             ae_ref, x_ref, uu3_ref):
  stall = stall_ref[...]
  dtt = dtt_ref[...]
  v = n2n_ref[...]
  adj = adj_ref[...]
  uu = jnp.zeros((_BLK, 8), jnp.float32)
  for h in range(_NH):
    whh = whbf_ref[:, h * _HH:(h + 1) * _HH]
    st = stall[:, h:h + 1]
    dt = dtt[h]
    esc = (ae_ref[0, h] * v[0] + ae_ref[1, h] * v[1]
           + ae_ref[2, h] * v[2] + ae_ref[3, h] * v[3])
    sc = _leaky(st + dt + esc)
    sc = jnp.where(adj > 0, sc, _NEG)
    m = jnp.max(sc, axis=1, keepdims=True)
    p = jnp.exp(sc - m)
    att = p / jnp.sum(p, axis=1, keepdims=True)
    o = _elu(jnp.dot(att.astype(jnp.bfloat16), whh,
                     preferred_element_type=jnp.float32))
    x_ref[:, h * _HH:(h + 1) * _HH] = o
    uu += jnp.dot(o, epg_ref[h * _HH:(h + 1) * _HH, :],
                  preferred_element_type=jnp.float32)
  uu3_ref[...] = uu


def _k4_body(uu3_ref, adj_ref, ep1b_ref, alpha_ref):
  u1 = uu3_ref[:, 0:1]
  u2r = jnp.reshape(uu3_ref[:, 1:2], (1, _N))
  s = _sig(u1 + u2r + ep1b_ref[0, 0])
  ns = jnp.sum(adj_ref[...] * s, axis=0, keepdims=True)
  alpha_ref[...] = jnp.reshape(1.0 + jnp.tanh(ns), (_N, 1))


def _k4c_body(x_ref, al_ref, uu3_ref, w2_ref, g2b_ref,
              wh2_ref, num_ref, den_ref):
  i = pl.program_id(0)
  al = al_ref[...]
  xn = al * x_ref[...]
  sg = _sig(al * uu3_ref[:, 2:3] + g2b_ref[0, 0])
  e = jnp.exp(sg)
  wh2_ref[...] = jnp.dot(xn, w2_ref[...], preferred_element_type=jnp.float32)

  @pl.when(i == 0)
  def _():
    num_ref[...] = jnp.zeros_like(num_ref)
    den_ref[...] = jnp.zeros_like(den_ref)

  num_ref[...] += jnp.sum(e * xn, axis=0, keepdims=True)
  den_ref[...] += jnp.sum(e, axis=0, keepdims=True)


def _k5_body(wh2f_ref, wh2b_ref, adj_ref, n2n_ref, uu3f_ref, uu3b_ref,
             asrc2_ref, adst2_ref, we_ref, ae2_ref, ep1b_ref, out_ref):
  wh2 = wh2f_ref[...]
  st2 = jnp.dot(wh2b_ref[...], asrc2_ref[...],
                preferred_element_type=jnp.float32)
  dt2 = jnp.reshape(jnp.dot(wh2, adst2_ref[...],
                            preferred_element_type=jnp.float32), (1, _N))
  u1 = uu3b_ref[:, 0:1]
  u2r = jnp.reshape(uu3f_ref[:, 1:2], (1, _N))
  s = _sig(u1 + u2r + ep1b_ref[0, 0])
  v = n2n_ref[...]
  acc = jnp.zeros((_BLK, _N), jnp.float32)
  for k in range(_NH):
    t = (we_ref[0, k] * v[0] + we_ref[1, k] * v[1]
         + we_ref[2, k] * v[2] + we_ref[3, k] * v[3])
    acc += ae2_ref[0, k] * _elu(t)
  sc = _leaky(st2 + dt2 + s * acc)
  sc = jnp.where(adj_ref[...] > 0, sc, _NEG)
  m = jnp.max(sc, axis=1, keepdims=True)
  p = jnp.exp(sc - m)
  att = p / jnp.sum(p, axis=1, keepdims=True)
  out_ref[...] = jnp.dot(att, wh2, preferred_element_type=jnp.float32)


def _k6_body(x2_ref, adj_ref, ep2w_ref, g3w_ref, n0_ref, d1_ref, n1_ref,
             d2_ref, wfa_ref, wfb_ref, wfc_ref, wba_ref, wbb_ref, wbc_ref,
             bf_ref, bb_ref, w2f_ref, w2b_ref, b2f_ref, b2b_ref,
             fcw_ref, fcb_ref, ep2b_ref, g3b_ref, out_ref):
  x2 = x2_ref[...]
  uu = jnp.dot(x2, ep2w_ref[...], preferred_element_type=jnp.float32)
  u1 = uu[:, 0:1]
  u2r = jnp.reshape(uu[:, 1:2], (1, _N))
  s2 = _sig(u1 + u2r + ep2b_ref[0, 0])
  ns2 = jnp.sum(adj_ref[...] * s2, axis=0, keepdims=True)
  al2 = jnp.reshape(1.0 + jnp.tanh(ns2), (_N, 1))
  xn2 = al2 * x2
  sg = _sig(jnp.dot(xn2, g3w_ref[...], preferred_element_type=jnp.float32)
            + g3b_ref[0, 0])
  m = jnp.max(sg)
  p = jnp.exp(sg - m)
  g = p / jnp.sum(p)
  hs2 = jnp.sum(g * xn2, axis=0, keepdims=True)
  hs0 = n0_ref[...] / d1_ref[...]
  hs1 = n1_ref[...] / d2_ref[...]

  def gate(z):
    i_ = z[:, 0:128]
    f_ = z[:, 128:256]
    g_ = z[:, 256:384]
    o_ = z[:, 384:512]
    del f_
    c = _sig(i_) * jnp.tanh(g_)
    return _sig(o_) * jnp.tanh(c)

  def lin3(wa, wb, wc, b):
    return (jnp.dot(hs0, wa, preferred_element_type=jnp.float32)
            + jnp.dot(hs1, wb, preferred_element_type=jnp.float32)
            + jnp.dot(hs2, wc, preferred_element_type=jnp.float32)
            + b)

  hf = gate(lin3(wfa_ref[...], wfb_ref[...], wfc_ref[...], bf_ref[...]))
  hb = gate(lin3(wba_ref[...], wbb_ref[...], wbc_ref[...], bb_ref[...]))

  def lin2(w, b):
    return (jnp.dot(hf, w[0:128, :], preferred_element_type=jnp.float32)
            + jnp.dot(hb, w[128:256, :], preferred_element_type=jnp.float32)
            + b)

  hf2 = gate(lin2(w2f_ref[...], b2f_ref[...]))
  hb2 = gate(lin2(w2b_ref[...], b2b_ref[...]))
  fw = fcw_ref[...]
  logits = (jnp.dot(hf2, fw[0:128, :], preferred_element_type=jnp.float32)
            + jnp.dot(hb2, fw[128:256, :], preferred_element_type=jnp.float32)
            + fcb_ref[...])
  m2 = jnp.max(logits, axis=1, keepdims=True)
  e2 = jnp.exp(logits - m2)
  out_ref[...] = e2 / jnp.sum(e2, axis=1, keepdims=True)


def _vspec(block=None, index_map=None):
  if block is None:
    return pl.BlockSpec(memory_space=pltpu.ANY)
  return pl.BlockSpec(block, index_map)


_SMEM = pl.BlockSpec(memory_space=pltpu.SMEM)
_F32 = jnp.float32


def kernel(features, edge_index, edgesAttr, adjacency, node2node_features,
           params):
  del edge_index, edgesAttr
  P = params
  zero = lambda *_: (0, 0)

  w1cat = jnp.transpose(P['W1'], (1, 0, 2)).reshape(_HID, _D1)
  eye16 = jnp.eye(_NH, dtype=_F32)
  asrc_bd = (P['a_src1'][:, :, 0][:, :, None]
             * eye16[:, None, :]).reshape(_D1, _NH)
  adst_bd = (P['a_dst1'][:, :, 0][:, :, None]
             * eye16[:, None, :]).reshape(_D1, _NH)
  ae1 = jnp.transpose(P['a_e1'][:, :, 0])
  we1c = jnp.transpose(P['We1'][:, :, 0])
  ae2 = jnp.transpose(P['a_e2'])
  epg = jnp.concatenate(
      [P['ep1_w'][:_D1], P['ep1_w'][_D1:], P['g2_w'],
       jnp.zeros((_D1, 5), _F32)], axis=1)
  ep2w = jnp.concatenate([P['ep2_w'][:_HH], P['ep2_w'][_HH:]], axis=1)
  n2nT = jnp.transpose(node2node_features.reshape(_N, _N, 4), (2, 0, 1))
  g1b = P['g1_b'].reshape(1, 1)
  g2b = P['g2_b'].reshape(1, 1)
  g3b = P['g3_b'].reshape(1, 1)
  ep1b = P['ep1_b'].reshape(1, 1)
  ep2b = P['ep2_b'].reshape(1, 1)
  bf = P['l1f_b'].reshape(1, 512)
  bb = P['l1b_b'].reshape(1, 512)
  b2f = P['l2f_b'].reshape(1, 512)
  b2b = P['l2b_b'].reshape(1, 512)
  fcb = P['fc_b'].reshape(1, 2)
  wfa, wfb, wfc = (P['l1f_Wi'][:128], P['l1f_Wi'][128:128 + _D1],
                   P['l1f_Wi'][128 + _D1:])
  wba, wbb, wbc = (P['l1b_Wi'][:128], P['l1b_Wi'][128:128 + _D1],
                   P['l1b_Wi'][128 + _D1:])

  whbf, stall, dtall, hs0num, den1 = pl.pallas_call(
      _k1_body,
      grid=(_NB,),
      in_specs=[
          pl.BlockSpec((_BLK, _HID), lambda i: (i, 0)),
          pl.BlockSpec((_HID, _HID), zero),
          pl.BlockSpec((_HID, 1), zero),
          pl.BlockSpec((_HID, 1), zero),
          _SMEM,
          pl.BlockSpec((_HID, _D1), zero),
          pl.BlockSpec((_D1, _NH), zero),
          pl.BlockSpec((_D1, _NH), zero),
      ],
      out_specs=[
          pl.BlockSpec((_BLK, _D1), lambda i: (i, 0)),
          pl.BlockSpec((_BLK, _NH), lambda i: (i, 0)),
          pl.BlockSpec((_BLK, _NH), lambda i: (i, 0)),
          pl.BlockSpec((1, _HID), zero),
          pl.BlockSpec((1, 1), zero),
      ],
      out_shape=[
          jax.ShapeDtypeStruct((_N, _D1), jnp.bfloat16),
          jax.ShapeDtypeStruct((_N, _NH), _F32),
          jax.ShapeDtypeStruct((_N, _NH), _F32),
          jax.ShapeDtypeStruct((1, _HID), _F32),
          jax.ShapeDtypeStruct((1, 1), _F32),
      ],
  )(features, P['snal_W'], P['snal_a'], P['g1_w'], g1b, w1cat,
    asrc_bd, adst_bd)

  dtt = jnp.transpose(dtall).reshape(_NH, 1, _N)

  x, uu3 = pl.pallas_call(
      _k2_body,
      grid=(_NB,),
      in_specs=[
          pl.BlockSpec((_N, _D1), lambda i: (0, 0)),
          pl.BlockSpec((_BLK, _NH), lambda i: (i, 0)),
          pl.BlockSpec((_NH, 1, _N), lambda i: (0, 0, 0)),
          pl.BlockSpec((_BLK, _N), lambda i: (i, 0)),
          pl.BlockSpec((4, _BLK, _N), lambda i: (0, i, 0)),
          pl.BlockSpec((_D1, 8), lambda i: (0, 0)),
          _SMEM,
      ],
      out_specs=[
          pl.BlockSpec((_BLK, _D1), lambda i: (i, 0)),
          pl.BlockSpec((_BLK, 8), lambda i: (i, 0)),
      ],
      out_shape=[
          jax.ShapeDtypeStruct((_N, _D1), _F32),
          jax.ShapeDtypeStruct((_N, 8), _F32),
      ],
  )(whbf, stall, dtt, adjacency, n2nT, epg, ae1)

  alpha = pl.pallas_call(
      _k4_body,
      in_specs=[
          pl.BlockSpec((_N, 8), zero),
          pl.BlockSpec((_N, _N), zero),
          _SMEM,
      ],
      out_specs=pl.BlockSpec((_N, 1), zero),
      out_shape=jax.ShapeDtypeStruct((_N, 1), _F32),
  )(uu3, adjacency, ep1b)

  wh2, hs1num, den2 = pl.pallas_call(
      _k4c_body,
      grid=(_NB,),
      in_specs=[
          pl.BlockSpec((_BLK, _D1), lambda i: (i, 0)),
          pl.BlockSpec((_BLK, 1), lambda i: (i, 0)),
          pl.BlockSpec((_BLK, 8), lambda i: (i, 0)),
          pl.BlockSpec((_D1, _HH), lambda i: (0, 0)),
          _SMEM,
      ],
      out_specs=[
          pl.BlockSpec((_BLK, _HH), lambda i: (i, 0)),
          pl.BlockSpec((1, _D1), lambda i: (0, 0)),
          pl.BlockSpec((1, 1), lambda i: (0, 0)),
      ],
      out_shape=[
          jax.ShapeDtypeStruct((_N, _HH), _F32),
          jax.ShapeDtypeStruct((1, _D1), _F32),
          jax.ShapeDtypeStruct((1, 1), _F32),
      ],
  )(x, alpha, uu3, P['W2'], g2b)

  hout2 = pl.pallas_call(
      _k5_body,
      grid=(_NB,),
      in_specs=[
          pl.BlockSpec((_N, _HH), lambda i: (0, 0)),
          pl.BlockSpec((_BLK, _HH), lambda i: (i, 0)),
          pl.BlockSpec((_BLK, _N), lambda i: (i, 0)),
          pl.BlockSpec((4, _BLK, _N), lambda i: (0, i, 0)),
          pl.BlockSpec((_N, 8), lambda i: (0, 0)),
          pl.BlockSpec((_BLK, 8), lambda i: (i, 0)),
          pl.BlockSpec((_HH, 1), lambda i: (0, 0)),
          pl.BlockSpec((_HH, 1), lambda i: (0, 0)),
          _SMEM,
          _SMEM,
          _SMEM,
      ],
      out_specs=pl.BlockSpec((_BLK, _HH), lambda i: (i, 0)),
      out_shape=jax.ShapeDtypeStruct((_N, _HH), _F32),
  )(wh2, wh2, adjacency, n2nT, uu3, uu3, P['a_src2'], P['a_dst2'],
    we1c, ae2, ep1b)

  out = pl.pallas_call(
      _k6_body,
      in_specs=[
          pl.BlockSpec((_N, _HH), zero),
          pl.BlockSpec((_N, _N), zero),
          pl.BlockSpec((_HH, 2), zero),
          pl.BlockSpec((_HH, 1), zero),
          pl.BlockSpec((1, _HID), zero),
          pl.BlockSpec((1, 1), zero),
          pl.BlockSpec((1, _D1), zero),
          pl.BlockSpec((1, 1), zero),
          pl.BlockSpec((128, 512), zero),
          pl.BlockSpec((_D1, 512), zero),
          pl.BlockSpec((_HH, 512), zero),
          pl.BlockSpec((128, 512), zero),
          pl.BlockSpec((_D1, 512), zero),
          pl.BlockSpec((_HH, 512), zero),
          pl.BlockSpec((1, 512), zero),
          pl.BlockSpec((1, 512), zero),
          pl.BlockSpec((_HH, 512), zero),
          pl.BlockSpec((_HH, 512), zero),
          pl.BlockSpec((1, 512), zero),
          pl.BlockSpec((1, 512), zero),
          pl.BlockSpec((_HH, 2), zero),
          pl.BlockSpec((1, 2), zero),
          _SMEM,
          _SMEM,
      ],
      out_specs=pl.BlockSpec((1, 2), zero),
      out_shape=jax.ShapeDtypeStruct((1, 2), _F32),
  )(hout2, adjacency, ep2w, P['g3_w'], hs0num, den1, hs1num, den2,
    wfa, wfb, wfc, wba, wbb, wbc, bf, bb, P['l2f_Wi'], P['l2b_Wi'],
    b2f, b2b, P['fc_w'], fcb, ep2b, g3b)

  return out.reshape(2)


# speedup vs baseline: 15.7342x; 1.0669x over previous
import jax
import jax.numpy as jnp
from jax import lax
from jax.experimental import pallas as pl
from jax.experimental.pallas import tpu as pltpu

_N = 1024
_HID = 128
_NH = 16
_HH = 256
_D1 = _NH * _HH
_ALPHA = 0.2
_BLK = 256
_NB = _N // _BLK
_NEG = -1e9


def _sig(x):
  return 1.0 / (1.0 + jnp.exp(-x))


def _leaky(x):
  return jnp.where(x > 0, x, _ALPHA * x)


def _elu(x):
  return jnp.where(x > 0, x, jnp.exp(x) - 1.0)


def _k1_body(feat_ref, snw_ref, sna_ref, g1w_ref, g1b_ref, w1c_ref,
             asrc_ref, adst_ref, whb_ref, stall_ref, dtall_ref,
             num_ref, den_ref):
  i = pl.program_id(0)
  f = feat_ref[...]
  h0 = jnp.dot(f, snw_ref[...], preferred_element_type=jnp.float32)
  t = jnp.dot(h0, sna_ref[...], preferred_element_type=jnp.float32)
  h = _elu(_sig(_leaky(t)) * h0)
  sg = _sig(jnp.dot(h, g1w_ref[...], preferred_element_type=jnp.float32)
            + g1b_ref[0, 0])
  e = jnp.exp(sg)
  wh = jnp.dot(h, w1c_ref[...], preferred_element_type=jnp.float32)
  whb_ref[...] = wh.astype(jnp.bfloat16)
  stall_ref[...] = jnp.dot(wh, asrc_ref[...],
                           preferred_element_type=jnp.float32)
  dtall_ref[...] = jnp.dot(wh, adst_ref[...],
                           preferred_element_type=jnp.float32)

  @pl.when(i == 0)
  def _():
    num_ref[...] = jnp.zeros_like(num_ref)
    den_ref[...] = jnp.zeros_like(den_ref)

  num_ref[...] += jnp.sum(e * h, axis=0, keepdims=True)
  den_ref[...] += jnp.sum(e, axis=0, keepdims=True)


def _k2_body(whbf_ref, stall_ref, dtt_ref, adj_ref, n2n_ref, epg_ref,
             ae_ref, x_ref, uu3_ref):
  stall = stall_ref[...]
  dtt = dtt_ref[...]
  v = n2n_ref[...]
  adj = adj_ref[...]
  uu = jnp.zeros((_BLK, 8), jnp.float32)
  for h in range(_NH):
    whh = whbf_ref[:, h * _HH:(h + 1) * _HH]
    st = stall[:, h:h + 1]
    dt = dtt[h]
    esc = (ae_ref[0, h] * v[0] + ae_ref[1, h] * v[1]
           + ae_ref[2, h] * v[2] + ae_ref[3, h] * v[3])
    sc = _leaky(st + dt + esc)
    sc = jnp.where(adj > 0, sc, _NEG)
    m = jnp.max(sc, axis=1, keepdims=True)
    p = jnp.exp(sc - m)
    att = p / jnp.sum(p, axis=1, keepdims=True)
    o = _elu(jnp.dot(att.astype(jnp.bfloat16), whh,
                     preferred_element_type=jnp.float32))
    x_ref[:, h * _HH:(h + 1) * _HH] = o.astype(jnp.bfloat16)
    uu += jnp.dot(o, epg_ref[h * _HH:(h + 1) * _HH, :],
                  preferred_element_type=jnp.float32)
  uu3_ref[...] = uu


def _k4c_body(x_ref, adjc_ref, uu3f_ref, uu3b_ref, w2_ref, ep1b_ref, g2b_ref,
              wh2_ref, num_ref, den_ref):
  i = pl.program_id(0)
  u1 = uu3f_ref[:, 0:1]
  u2b = jnp.reshape(uu3b_ref[:, 1:2], (1, _BLK))
  s = _sig(u1 + u2b + ep1b_ref[0, 0])
  ns = jnp.sum(adjc_ref[...] * s, axis=0, keepdims=True)
  al = jnp.reshape(1.0 + jnp.tanh(ns), (_BLK, 1))
  xn = al * x_ref[...].astype(jnp.float32)
  sg = _sig(al * uu3b_ref[:, 2:3] + g2b_ref[0, 0])
  e = jnp.exp(sg)
  wh2_ref[...] = jnp.dot(xn.astype(jnp.bfloat16), w2_ref[...],
                         preferred_element_type=jnp.float32)

  @pl.when(i == 0)
  def _():
    num_ref[...] = jnp.zeros_like(num_ref)
    den_ref[...] = jnp.zeros_like(den_ref)

  num_ref[...] += jnp.sum(e * xn, axis=0, keepdims=True)
  den_ref[...] += jnp.sum(e, axis=0, keepdims=True)


def _k5_body(wh2f_ref, wh2b_ref, adj_ref, n2n_ref, uu3f_ref, uu3b_ref,
             asrc2_ref, adst2_ref, we_ref, ae2_ref, ep1b_ref, out_ref):
  wh2 = wh2f_ref[...]
  st2 = jnp.dot(wh2b_ref[...], asrc2_ref[...],
                preferred_element_type=jnp.float32)
  dt2 = jnp.reshape(jnp.dot(wh2, adst2_ref[...],
                            preferred_element_type=jnp.float32), (1, _N))
  u1 = uu3b_ref[:, 0:1]
  u2r = jnp.reshape(uu3f_ref[:, 1:2], (1, _N))
  s = _sig(u1 + u2r + ep1b_ref[0, 0])
  v = n2n_ref[...]
  acc = jnp.zeros((_BLK, _N), jnp.float32)
  for k in range(_NH):
    t = (we_ref[0, k] * v[0] + we_ref[1, k] * v[1]
         + we_ref[2, k] * v[2] + we_ref[3, k] * v[3])
    acc += ae2_ref[0, k] * _elu(t)
  sc = _leaky(st2 + dt2 + s * acc)
  sc = jnp.where(adj_ref[...] > 0, sc, _NEG)
  m = jnp.max(sc, axis=1, keepdims=True)
  p = jnp.exp(sc - m)
  att = p / jnp.sum(p, axis=1, keepdims=True)
  out_ref[...] = jnp.dot(att.astype(jnp.bfloat16), wh2.astype(jnp.bfloat16),
                         preferred_element_type=jnp.float32)


def _k6_body(x2_ref, adj_ref, ep2w_ref, g3w_ref, n0_ref, d1_ref, n1_ref,
             d2_ref, wfa_ref, wfb_ref, wfc_ref, wba_ref, wbb_ref, wbc_ref,
             bf_ref, bb_ref, w2f_ref, w2b_ref, b2f_ref, b2b_ref,
             fcw_ref, fcb_ref, ep2b_ref, g3b_ref, out_ref):
  x2 = x2_ref[...]
  uu = jnp.dot(x2, ep2w_ref[...], preferred_element_type=jnp.float32)
  u1 = uu[:, 0:1]
  u2r = jnp.reshape(uu[:, 1:2], (1, _N))
  s2 = _sig(u1 + u2r + ep2b_ref[0, 0])
  ns2 = jnp.sum(adj_ref[...] * s2, axis=0, keepdims=True)
  al2 = jnp.reshape(1.0 + jnp.tanh(ns2), (_N, 1))
  xn2 = al2 * x2
  sg = _sig(jnp.dot(xn2, g3w_ref[...], preferred_element_type=jnp.float32)
            + g3b_ref[0, 0])
  m = jnp.max(sg)
  p = jnp.exp(sg - m)
  g = p / jnp.sum(p)
  hs2 = jnp.sum(g * xn2, axis=0, keepdims=True)
  hs0 = n0_ref[...] / d1_ref[...]
  hs1 = n1_ref[...] / d2_ref[...]

  def gate(z):
    i_ = z[:, 0:128]
    f_ = z[:, 128:256]
    g_ = z[:, 256:384]
    o_ = z[:, 384:512]
    del f_
    c = _sig(i_) * jnp.tanh(g_)
    return _sig(o_) * jnp.tanh(c)

  h0b = hs0.astype(jnp.bfloat16)
  h1b = hs1.astype(jnp.bfloat16)
  h2b = hs2.astype(jnp.bfloat16)

  def lin3(wa, wb, wc, b):
    return (jnp.dot(h0b, wa, preferred_element_type=jnp.float32)
            + jnp.dot(h1b, wb, preferred_element_type=jnp.float32)
            + jnp.dot(h2b, wc, preferred_element_type=jnp.float32)
            + b)

  hf = gate(lin3(wfa_ref[...], wfb_ref[...], wfc_ref[...], bf_ref[...]))
  hb = gate(lin3(wba_ref[...], wbb_ref[...], wbc_ref[...], bb_ref[...]))

  def lin2(w, b):
    hfb = hf.astype(jnp.bfloat16)
    hbb = hb.astype(jnp.bfloat16)
    return (jnp.dot(hfb, w[0:128, :], preferred_element_type=jnp.float32)
            + jnp.dot(hbb, w[128:256, :], preferred_element_type=jnp.float32)
            + b)

  hf2 = gate(lin2(w2f_ref[...], b2f_ref[...]))
  hb2 = gate(lin2(w2b_ref[...], b2b_ref[...]))
  fw = fcw_ref[...]
  logits = (jnp.dot(hf2, fw[0:128, :], preferred_element_type=jnp.float32)
            + jnp.dot(hb2, fw[128:256, :], preferred_element_type=jnp.float32)
            + fcb_ref[...])
  m2 = jnp.max(logits, axis=1, keepdims=True)
  e2 = jnp.exp(logits - m2)
  out_ref[...] = e2 / jnp.sum(e2, axis=1, keepdims=True)


def _vspec(block=None, index_map=None):
  if block is None:
    return pl.BlockSpec(memory_space=pltpu.ANY)
  return pl.BlockSpec(block, index_map)


_SMEM = pl.BlockSpec(memory_space=pltpu.SMEM)
_F32 = jnp.float32


def kernel(features, edge_index, edgesAttr, adjacency, node2node_features,
           params):
  del edge_index, edgesAttr
  P = params
  zero = lambda *_: (0, 0)

  w1cat = jnp.transpose(P['W1'], (1, 0, 2)).reshape(_HID, _D1)
  eye16 = jnp.eye(_NH, dtype=_F32)
  asrc_bd = (P['a_src1'][:, :, 0][:, :, None]
             * eye16[:, None, :]).reshape(_D1, _NH)
  adst_bd = (P['a_dst1'][:, :, 0][:, :, None]
             * eye16[:, None, :]).reshape(_D1, _NH)
  ae1 = jnp.transpose(P['a_e1'][:, :, 0])
  we1c = jnp.transpose(P['We1'][:, :, 0])
  ae2 = jnp.transpose(P['a_e2'])
  epg = jnp.concatenate(
      [P['ep1_w'][:_D1], P['ep1_w'][_D1:], P['g2_w'],
       jnp.zeros((_D1, 5), _F32)], axis=1)
  ep2w = jnp.concatenate([P['ep2_w'][:_HH], P['ep2_w'][_HH:]], axis=1)
  n2nT = jnp.transpose(node2node_features.reshape(_N, _N, 4), (2, 0, 1))
  g1b = P['g1_b'].reshape(1, 1)
  g2b = P['g2_b'].reshape(1, 1)
  g3b = P['g3_b'].reshape(1, 1)
  ep1b = P['ep1_b'].reshape(1, 1)
  ep2b = P['ep2_b'].reshape(1, 1)
  bf = P['l1f_b'].reshape(1, 512)
  bb = P['l1b_b'].reshape(1, 512)
  b2f = P['l2f_b'].reshape(1, 512)
  b2b = P['l2b_b'].reshape(1, 512)
  fcb = P['fc_b'].reshape(1, 2)
  bf16 = jnp.bfloat16
  w2bf = P['W2'].astype(bf16)
  l1f = P['l1f_Wi'].astype(bf16)
  l1b = P['l1b_Wi'].astype(bf16)
  wfa, wfb, wfc = l1f[:128], l1f[128:128 + _D1], l1f[128 + _D1:]
  wba, wbb, wbc = l1b[:128], l1b[128:128 + _D1], l1b[128 + _D1:]
  w2fbf = P['l2f_Wi'].astype(bf16)
  w2bbf = P['l2b_Wi'].astype(bf16)

  whbf, stall, dtall, hs0num, den1 = pl.pallas_call(
      _k1_body,
      grid=(_NB,),
      in_specs=[
          pl.BlockSpec((_BLK, _HID), lambda i: (i, 0)),
          pl.BlockSpec((_HID, _HID), zero),
          pl.BlockSpec((_HID, 1), zero),
          pl.BlockSpec((_HID, 1), zero),
          _SMEM,
          pl.BlockSpec((_HID, _D1), zero),
          pl.BlockSpec((_D1, _NH), zero),
          pl.BlockSpec((_D1, _NH), zero),
      ],
      out_specs=[
          pl.BlockSpec((_BLK, _D1), lambda i: (i, 0)),
          pl.BlockSpec((_BLK, _NH), lambda i: (i, 0)),
          pl.BlockSpec((_BLK, _NH), lambda i: (i, 0)),
          pl.BlockSpec((1, _HID), zero),
          pl.BlockSpec((1, 1), zero),
      ],
      out_shape=[
          jax.ShapeDtypeStruct((_N, _D1), jnp.bfloat16),
          jax.ShapeDtypeStruct((_N, _NH), _F32),
          jax.ShapeDtypeStruct((_N, _NH), _F32),
          jax.ShapeDtypeStruct((1, _HID), _F32),
          jax.ShapeDtypeStruct((1, 1), _F32),
      ],
  )(features, P['snal_W'], P['snal_a'], P['g1_w'], g1b, w1cat,
    asrc_bd, adst_bd)

  dtt = jnp.transpose(dtall).reshape(_NH, 1, _N)

  x, uu3 = pl.pallas_call(
      _k2_body,
      grid=(_NB,),
      in_specs=[
          pl.BlockSpec((_N, _D1), lambda i: (0, 0)),
          pl.BlockSpec((_BLK, _NH), lambda i: (i, 0)),
          pl.BlockSpec((_NH, 1, _N), lambda i: (0, 0, 0)),
          pl.BlockSpec((_BLK, _N), lambda i: (i, 0)),
          pl.BlockSpec((4, _BLK, _N), lambda i: (0, i, 0)),
          pl.BlockSpec((_D1, 8), lambda i: (0, 0)),
          _SMEM,
      ],
      out_specs=[
          pl.BlockSpec((_BLK, _D1), lambda i: (i, 0)),
          pl.BlockSpec((_BLK, 8), lambda i: (i, 0)),
      ],
      out_shape=[
          jax.ShapeDtypeStruct((_N, _D1), jnp.bfloat16),
          jax.ShapeDtypeStruct((_N, 8), _F32),
      ],
  )(whbf, stall, dtt, adjacency, n2nT, epg, ae1)

  wh2, hs1num, den2 = pl.pallas_call(
      _k4c_body,
      grid=(_NB,),
      in_specs=[
          pl.BlockSpec((_BLK, _D1), lambda i: (i, 0)),
          pl.BlockSpec((_N, _BLK), lambda i: (0, i)),
          pl.BlockSpec((_N, 8), lambda i: (0, 0)),
          pl.BlockSpec((_BLK, 8), lambda i: (i, 0)),
          pl.BlockSpec((_D1, _HH), lambda i: (0, 0)),
          _SMEM,
          _SMEM,
      ],
      out_specs=[
          pl.BlockSpec((_BLK, _HH), lambda i: (i, 0)),
          pl.BlockSpec((1, _D1), lambda i: (0, 0)),
          pl.BlockSpec((1, 1), lambda i: (0, 0)),
      ],
      out_shape=[
          jax.ShapeDtypeStruct((_N, _HH), _F32),
          jax.ShapeDtypeStruct((1, _D1), _F32),
          jax.ShapeDtypeStruct((1, 1), _F32),
      ],
  )(x, adjacency, uu3, uu3, w2bf, ep1b, g2b)

  hout2 = pl.pallas_call(
      _k5_body,
      grid=(_NB,),
      in_specs=[
          pl.BlockSpec((_N, _HH), lambda i: (0, 0)),
          pl.BlockSpec((_BLK, _HH), lambda i: (i, 0)),
          pl.BlockSpec((_BLK, _N), lambda i: (i, 0)),
          pl.BlockSpec((4, _BLK, _N), lambda i: (0, i, 0)),
          pl.BlockSpec((_N, 8), lambda i: (0, 0)),
          pl.BlockSpec((_BLK, 8), lambda i: (i, 0)),
          pl.BlockSpec((_HH, 1), lambda i: (0, 0)),
          pl.BlockSpec((_HH, 1), lambda i: (0, 0)),
          _SMEM,
          _SMEM,
          _SMEM,
      ],
      out_specs=pl.BlockSpec((_BLK, _HH), lambda i: (i, 0)),
      out_shape=jax.ShapeDtypeStruct((_N, _HH), _F32),
  )(wh2, wh2, adjacency, n2nT, uu3, uu3, P['a_src2'], P['a_dst2'],
    we1c, ae2, ep1b)

  out = pl.pallas_call(
      _k6_body,
      in_specs=[
          pl.BlockSpec((_N, _HH), zero),
          pl.BlockSpec((_N, _N), zero),
          pl.BlockSpec((_HH, 2), zero),
          pl.BlockSpec((_HH, 1), zero),
          pl.BlockSpec((1, _HID), zero),
          pl.BlockSpec((1, 1), zero),
          pl.BlockSpec((1, _D1), zero),
          pl.BlockSpec((1, 1), zero),
          pl.BlockSpec((128, 512), zero),
          pl.BlockSpec((_D1, 512), zero),
          pl.BlockSpec((_HH, 512), zero),
          pl.BlockSpec((128, 512), zero),
          pl.BlockSpec((_D1, 512), zero),
          pl.BlockSpec((_HH, 512), zero),
          pl.BlockSpec((1, 512), zero),
          pl.BlockSpec((1, 512), zero),
          pl.BlockSpec((_HH, 512), zero),
          pl.BlockSpec((_HH, 512), zero),
          pl.BlockSpec((1, 512), zero),
          pl.BlockSpec((1, 512), zero),
          pl.BlockSpec((_HH, 2), zero),
          pl.BlockSpec((1, 2), zero),
          _SMEM,
          _SMEM,
      ],
      out_specs=pl.BlockSpec((1, 2), zero),
      out_shape=jax.ShapeDtypeStruct((1, 2), _F32),
  )(hout2, adjacency, ep2w, P['g3_w'], hs0num, den1, hs1num, den2,
    wfa, wfb, wfc, wba, wbb, wbc, bf, bb, w2fbf, w2bbf,
    b2f, b2b, P['fc_w'], fcb, ep2b, g3b)

  return out.reshape(2)
